# baseline (speedup 1.0000x reference)
import functools

import jax
import jax.numpy as jnp
from jax import lax
from jax.experimental import pallas as pl
from jax.experimental.pallas import tpu as pltpu
from jax.experimental.pallas import tpu_sc as plsc

_NSHIFT = 8
_GROWS = 128


def _sc_build_windows(seq: int, d: int, num_rel: int, win_rows: int):
    max_len = (num_rel + 1) // 2
    win_start = max_len - seq
    info = plsc.get_sparse_core_info()
    nc, ns, nl = info.num_cores, info.num_subcores, info.num_lanes
    nw = nc * ns
    n_sub = -(-win_rows // _GROWS)
    tasks_per_w = -(-(_NSHIFT * n_sub) // nw)
    assert _NSHIFT * n_sub == nw * tasks_per_w and d % nl == 0
    tail = win_rows - (n_sub - 1) * _GROWS

    mesh = plsc.VectorSubcoreMesh(core_axis_name="c", subcore_axis_name="s")

    @functools.partial(
        pl.kernel,
        mesh=mesh,
        out_type=jax.ShapeDtypeStruct((_NSHIFT, win_rows, d), jnp.float32),
        scratch_types=[
            pltpu.VMEM((_GROWS,), jnp.int32),
            pltpu.VMEM((_GROWS, d), jnp.float32),
            pltpu.SemaphoreType.DMA,
        ],
    )
    def body(table_hbm, win8_hbm, idx_v, rows_v, gsem):
        cid = lax.axis_index("c")
        sid = lax.axis_index("s")
        wid = sid * nc + cid
        for jj in range(tasks_per_w):
            t = wid * tasks_per_w + jj
            s = t // n_sub
            g = t - s * n_sub
            gr0 = win_start + s + g * _GROWS
            for gg in range(_GROWS // nl):
                idx_v[pl.ds(gg * nl, nl)] = gr0 + gg * nl + lax.iota(jnp.int32, nl)
            pltpu.async_copy(table_hbm.at[idx_v], rows_v, gsem).wait()

            @pl.when(g < n_sub - 1)
            def _full():
                pltpu.sync_copy(
                    rows_v,
                    win8_hbm.at[s, pl.ds(pl.multiple_of(g * _GROWS, _GROWS), _GROWS), :],
                )

            @pl.when(g == n_sub - 1)
            def _tail():
                pltpu.sync_copy(
                    rows_v.at[pl.ds(0, tail)],
                    win8_hbm.at[s, pl.ds((n_sub - 1) * _GROWS, tail), :],
                )

    return body


def _tc_broadcast(seq: int, d: int, win_rows: int):
    blk = 2 * _NSHIFT
    nblk = seq // blk

    def body(win8_ref, out_ref):
        q = pl.program_id(0)
        for r in range(blk):
            s = _NSHIFT - 1 - (r % _NSHIFT)
            off = pl.multiple_of(
                seq - _NSHIFT - blk * q - _NSHIFT * (r // _NSHIFT), _NSHIFT
            )
            out_ref[r] = win8_ref[s, pl.ds(off, seq), :]

    return pl.pallas_call(
        body,
        grid=(nblk,),
        out_shape=jax.ShapeDtypeStruct((seq, seq, d), jnp.float32),
        in_specs=[
            pl.BlockSpec((_NSHIFT, win_rows, d), lambda q: (0, 0, 0)),
        ],
        out_specs=pl.BlockSpec((blk, seq, d), lambda q: (q, 0, 0)),
        compiler_params=pltpu.CompilerParams(
            dimension_semantics=("arbitrary",),
            vmem_limit_bytes=100 * 1024 * 1024,
        ),
    )


def kernel(x, relative_embeddings):
    seq = x.shape[0]
    d = relative_embeddings.shape[1]
    num_rel = relative_embeddings.shape[0]
    win_rows = 2 * seq - _NSHIFT

    win8 = _sc_build_windows(seq, d, num_rel, win_rows)(relative_embeddings)
    return _tc_broadcast(seq, d, win_rows)(win8)


# speedup vs baseline: 1.0165x; 1.0165x over previous
import functools

import jax
import jax.numpy as jnp
from jax import lax
from jax.experimental import pallas as pl
from jax.experimental.pallas import tpu as pltpu
from jax.experimental.pallas import tpu_sc as plsc

_NSHIFT = 8
_GROWS = 128


def _sc_build_windows(seq: int, d: int, num_rel: int, win_rows: int):
    max_len = (num_rel + 1) // 2
    win_start = max_len - seq
    info = plsc.get_sparse_core_info()
    nc, ns, nl = info.num_cores, info.num_subcores, info.num_lanes
    nw = nc * ns
    n_sub = -(-win_rows // _GROWS)
    tasks_per_w = -(-(_NSHIFT * n_sub) // nw)
    assert _NSHIFT * n_sub == nw * tasks_per_w and d % nl == 0
    tail = win_rows - (n_sub - 1) * _GROWS

    mesh = plsc.VectorSubcoreMesh(core_axis_name="c", subcore_axis_name="s")

    @functools.partial(
        pl.kernel,
        mesh=mesh,
        out_type=jax.ShapeDtypeStruct((_NSHIFT, win_rows, d), jnp.float32),
        scratch_types=[
            [pltpu.VMEM((_GROWS,), jnp.int32) for _ in range(2)],
            [pltpu.VMEM((_GROWS, d), jnp.float32) for _ in range(2)],
            pltpu.SemaphoreType.DMA,
            pltpu.SemaphoreType.DMA,
        ],
    )
    def body(table_hbm, win8_hbm, idx_vs, rows_vs, gsem, wsem):
        cid = lax.axis_index("c")
        sid = lax.axis_index("s")
        wid = sid * nc + cid
        assert tasks_per_w == 2
        gathers, writes = [], []
        for jj in range(tasks_per_w):
            t = wid * tasks_per_w + jj
            s = t // n_sub
            g = t - s * n_sub
            gr0 = win_start + s + g * _GROWS
            for gg in range(_GROWS // nl):
                idx_vs[jj][pl.ds(gg * nl, nl)] = gr0 + gg * nl + lax.iota(jnp.int32, nl)
            c = pltpu.make_async_copy(table_hbm.at[idx_vs[jj]], rows_vs[jj], gsem)
            c.start()
            gathers.append((c, s, g))
        for jj in range(tasks_per_w):
            c, s, g = gathers[jj]
            c.wait()

            @pl.when(g < n_sub - 1)
            def _full():
                w = pltpu.make_async_copy(
                    rows_vs[jj],
                    win8_hbm.at[s, pl.ds(pl.multiple_of(g * _GROWS, _GROWS), _GROWS), :],
                    wsem,
                )
                w.start()

            @pl.when(g == n_sub - 1)
            def _tail():
                w = pltpu.make_async_copy(
                    rows_vs[jj].at[pl.ds(0, tail)],
                    win8_hbm.at[s, pl.ds((n_sub - 1) * _GROWS, tail), :],
                    wsem,
                )
                w.start()

        for jj in range(tasks_per_w):
            _, s, g = gathers[jj]

            @pl.when(g < n_sub - 1)
            def _wfull():
                pltpu.make_async_copy(
                    rows_vs[jj],
                    win8_hbm.at[s, pl.ds(pl.multiple_of(g * _GROWS, _GROWS), _GROWS), :],
                    wsem,
                ).wait()

            @pl.when(g == n_sub - 1)
            def _wtail():
                pltpu.make_async_copy(
                    rows_vs[jj].at[pl.ds(0, tail)],
                    win8_hbm.at[s, pl.ds((n_sub - 1) * _GROWS, tail), :],
                    wsem,
                ).wait()

    return body


def _tc_broadcast(seq: int, d: int, win_rows: int):
    blk = _NSHIFT
    nblk = seq // blk

    def body(win8_ref, out_ref):
        q = pl.program_id(0)
        for r in range(blk):
            s = _NSHIFT - 1 - (r % _NSHIFT)
            off = pl.multiple_of(
                seq - _NSHIFT - blk * q - _NSHIFT * (r // _NSHIFT), _NSHIFT
            )
            out_ref[r] = win8_ref[s, pl.ds(off, seq), :]

    return pl.pallas_call(
        body,
        grid=(nblk,),
        out_shape=jax.ShapeDtypeStruct((seq, seq, d), jnp.float32),
        in_specs=[
            pl.BlockSpec((_NSHIFT, win_rows, d), lambda q: (0, 0, 0)),
        ],
        out_specs=pl.BlockSpec((blk, seq, d), lambda q: (q, 0, 0)),
        compiler_params=pltpu.CompilerParams(
            dimension_semantics=("arbitrary",),
            vmem_limit_bytes=100 * 1024 * 1024,
        ),
    )


def kernel(x, relative_embeddings):
    seq = x.shape[0]
    d = relative_embeddings.shape[1]
    num_rel = relative_embeddings.shape[0]
    win_rows = 2 * seq - _NSHIFT

    win8 = _sc_build_windows(seq, d, num_rel, win_rows)(relative_embeddings)
    return _tc_broadcast(seq, d, win_rows)(win8)
